# baseline (speedup 1.0000x reference)
import jax
import jax.numpy as jnp
from jax.experimental import pallas as pl

N = 256
F = 255
E = 65536
HIDDEN = 2048
OUT_DIM = 32640


_EC = 2048
_NEC = E // _EC


def _c_kernel(src_ref, dst_ref, c_ref):
    i = pl.program_id(0)
    s = src_ref[0]
    d = dst_ref[0]
    rows = jax.lax.broadcasted_iota(jnp.int32, (N, _EC), 0)
    ohs = (rows == s).astype(jnp.float32)
    ohd = (rows == d).astype(jnp.float32)
    blk = jax.lax.dot_general(
        ohd, ohs, (((1,), (1,)), ((), ())),
        preferred_element_type=jnp.float32)

    @pl.when(i == 0)
    def _():
        rr = jax.lax.broadcasted_iota(jnp.int32, (N, N), 0)
        cc = jax.lax.broadcasted_iota(jnp.int32, (N, N), 1)
        c_ref[...] = blk + (rr == cc).astype(jnp.float32)

    @pl.when(i > 0)
    def _():
        c_ref[...] += blk


def _build_counts(src3, dst3):
    return pl.pallas_call(
        _c_kernel,
        grid=(_NEC,),
        in_specs=[
            pl.BlockSpec((1, 1, _EC), lambda i: (i, 0, 0)),
            pl.BlockSpec((1, 1, _EC), lambda i: (i, 0, 0)),
        ],
        out_specs=pl.BlockSpec((N, N), lambda i: (0, 0)),
        out_shape=jax.ShapeDtypeStruct((N, N), jnp.float32),
    )(src3, dst3)


def _gat_layer(h_in, C, mask, W, a_s_row, a_d_col, b_row):
    h = jnp.dot(h_in, W, preferred_element_type=jnp.float32)
    al_s = jax.lax.dot_general(
        a_s_row, h, (((1,), (1,)), ((), ())),
        preferred_element_type=jnp.float32)
    al_d = jnp.dot(h, a_d_col, preferred_element_type=jnp.float32)
    e = al_d + al_s
    e = jnp.where(e >= 0, e, 0.2 * e)
    em = jnp.where(mask, e, -1e30)
    m = jnp.max(em, axis=1, keepdims=True)
    p = jnp.exp(em - m) * C
    denom = jnp.sum(p, axis=1, keepdims=True)
    P = p / (denom + 1e-16)
    return jnp.dot(P, h, preferred_element_type=jnp.float32) + b_row


def _gat3_kernel(x_ref, c_ref,
                 w1_ref, as1_ref, ad1_ref, b1_ref,
                 w2_ref, as2_ref, ad2_ref, b2_ref,
                 w3_ref, as3_ref, ad3_ref, b3_ref,
                 out_ref):
    C = c_ref[...]
    mask = C > 0
    h = x_ref[...]
    h = _gat_layer(h, C, mask, w1_ref[...], as1_ref[...], ad1_ref[...],
                   b1_ref[...])
    h = _gat_layer(h, C, mask, w2_ref[...], as2_ref[...], ad2_ref[...],
                   b2_ref[...])
    h = _gat_layer(h, C, mask, w3_ref[...], as3_ref[...], ad3_ref[...],
                   b3_ref[...])
    out_ref[...] = jnp.maximum(h[:, :F], 0.0)


def _run_gat3(xp, C, layer_params):
    flat = []
    for (Wp, a_s, a_d, b) in layer_params:
        flat += [Wp, a_s, a_d, b]
    return pl.pallas_call(
        _gat3_kernel,
        out_shape=jax.ShapeDtypeStruct((N, F), jnp.float32),
    )(xp, C, *flat)


_K1 = 3840
_N1 = 512
_NK1 = (N * F) // _K1
_NN1 = HIDDEN // _N1 // 2


def _mv1_kernel(y_ref, wa_ref, wb_ref, ba_ref, bb_ref, oa_ref, ob_ref):
    k = pl.program_id(1)
    pa = jnp.dot(y_ref[...], wa_ref[...], preferred_element_type=jnp.float32)
    pb = jnp.dot(y_ref[...], wb_ref[...], preferred_element_type=jnp.float32)

    @pl.when(k == 0)
    def _():
        oa_ref[...] = pa
        ob_ref[...] = pb

    @pl.when(k > 0)
    def _():
        oa_ref[...] += pa
        ob_ref[...] += pb

    @pl.when(k == _NK1 - 1)
    def _():
        oa_ref[...] = jnp.maximum(oa_ref[...] + ba_ref[...], 0.0)
        ob_ref[...] = jnp.maximum(ob_ref[...] + bb_ref[...], 0.0)


def _run_mv1(y0, W4, b4):
    oa, ob = pl.pallas_call(
        _mv1_kernel,
        grid=(_NN1, _NK1),
        in_specs=[
            pl.BlockSpec((1, _K1), lambda n, k: (0, k)),
            pl.BlockSpec((_K1, _N1), lambda n, k: (k, n)),
            pl.BlockSpec((_K1, _N1), lambda n, k: (k, n + _NN1)),
            pl.BlockSpec((1, _N1), lambda n, k: (0, n)),
            pl.BlockSpec((1, _N1), lambda n, k: (0, n + _NN1)),
        ],
        out_specs=[
            pl.BlockSpec((1, _N1), lambda n, k: (0, n)),
            pl.BlockSpec((1, _N1), lambda n, k: (0, n)),
        ],
        out_shape=[
            jax.ShapeDtypeStruct((1, HIDDEN // 2), jnp.float32),
            jax.ShapeDtypeStruct((1, HIDDEN // 2), jnp.float32),
        ],
    )(y0, W4, W4, b4, b4)
    return jnp.concatenate([oa, ob], axis=1)


_N2 = 1920
_NN2 = OUT_DIM // _N2
_K2 = HIDDEN // 2


def _mv2_kernel(ya_ref, yb_ref, wa_ref, wb_ref, b_ref, o_ref):
    o_ref[...] = (
        jnp.dot(ya_ref[...], wa_ref[...], preferred_element_type=jnp.float32)
        + jnp.dot(yb_ref[...], wb_ref[...],
                  preferred_element_type=jnp.float32)
        + b_ref[...])


def _run_mv2(y1, W5, b5):
    return pl.pallas_call(
        _mv2_kernel,
        grid=(_NN2,),
        in_specs=[
            pl.BlockSpec((1, _K2), lambda j: (0, 0)),
            pl.BlockSpec((1, _K2), lambda j: (0, 1)),
            pl.BlockSpec((_K2, _N2), lambda j: (0, j)),
            pl.BlockSpec((_K2, _N2), lambda j: (1, j)),
            pl.BlockSpec((1, _N2), lambda j: (0, j)),
        ],
        out_specs=pl.BlockSpec((1, _N2), lambda j: (0, j)),
        out_shape=jax.ShapeDtypeStruct((1, OUT_DIM), jnp.float32),
    )(y1, y1, W5, W5, b5)


def _pad_w(W):
    return jnp.pad(W, ((0, 1), (0, 1)))


def kernel(x, edge_index, W1, a_src1, a_dst1, b1, W2, a_src2, a_dst2, b2,
           W3, a_src3, a_dst3, b3, W4, b4, W5, b5):
    ei = edge_index.astype(jnp.int32)
    src3 = ei[0].reshape(_NEC, 1, _EC)
    dst3 = ei[1].reshape(_NEC, 1, _EC)
    C = _build_counts(src3, dst3)

    xp = jnp.pad(x, ((0, 0), (0, 1)))
    layer_params = []
    for (W, a_s, a_d, b) in ((W1, a_src1, a_dst1, b1),
                             (W2, a_src2, a_dst2, b2),
                             (W3, a_src3, a_dst3, b3)):
        layer_params.append((
            _pad_w(W),
            jnp.pad(a_s, (0, 1)).reshape(1, N),
            jnp.pad(a_d, (0, 1)).reshape(N, 1),
            jnp.pad(b, (0, 1)).reshape(1, N),
        ))

    g3r = _run_gat3(xp, C, layer_params)
    y0 = g3r.reshape(1, N * F)
    y1 = _run_mv1(y0, W4, b4.reshape(1, HIDDEN))
    y2 = _run_mv2(y1, W5, b5.reshape(1, OUT_DIM))
    return y2.reshape(OUT_DIM)


# speedup vs baseline: 1.0160x; 1.0160x over previous
import functools

import jax
import jax.numpy as jnp
from jax import lax
from jax.experimental import pallas as pl
from jax.experimental.pallas import tpu as pltpu
from jax.experimental.pallas import tpu_sc as plsc

N = 256
F = 255
E = 65536
HIDDEN = 2048
OUT_DIM = 32640


_NC = 2
_NS = 16
_NW = _NC * _NS
_EPW = E // _NW
_CSZ = N * N
_ZSL = _CSZ // _NS
_NROW = _EPW // 128


def _sc_counts_body(src_hbm, dst_hbm, out_hbm,
                    src_v, dst_v, idx_v, val_v, zer_v, c_sh):
    cid = lax.axis_index("c")
    sid = lax.axis_index("s")
    wid = sid * _NC + cid
    base = wid * _EPW
    pltpu.sync_copy(src_hbm.at[pl.ds(base, _EPW)], src_v)
    pltpu.sync_copy(dst_hbm.at[pl.ds(base, _EPW)], dst_v)

    zeros = jnp.zeros((16,), jnp.float32)
    ones = jnp.ones((16,), jnp.float32)
    for j in range(_ZSL // 16):
        zer_v[pl.ds(j * 16, 16)] = zeros
    for k in range(8):
        val_v[pl.ds(k * 16, 16)] = ones
    for j in range(_NROW):
        for k in range(8):
            s = pl.ds((j * 8 + k) * 16, 16)
            idx_v[j, pl.ds(k * 16, 16)] = dst_v[s] * 256 + src_v[s]

    pltpu.sync_copy(zer_v, c_sh.at[pl.ds(sid * _ZSL, _ZSL)])
    plsc.subcore_barrier()
    for j in range(_NROW):
        pltpu.sync_copy(val_v, c_sh.at[idx_v.at[j]], add=True)
    plsc.subcore_barrier()
    pltpu.sync_copy(c_sh.at[pl.ds(sid * _ZSL, _ZSL)],
                    out_hbm.at[cid, pl.ds(sid * _ZSL, _ZSL)])


_sc_counts = functools.partial(
    pl.kernel,
    out_type=jax.ShapeDtypeStruct((_NC, _CSZ), jnp.float32),
    mesh=plsc.VectorSubcoreMesh(core_axis_name="c", subcore_axis_name="s"),
    scratch_types=[
        pltpu.VMEM((_EPW,), jnp.int32),
        pltpu.VMEM((_EPW,), jnp.int32),
        pltpu.VMEM((_NROW, 128), jnp.int32),
        pltpu.VMEM((128,), jnp.float32),
        pltpu.VMEM((_ZSL,), jnp.float32),
        pltpu.VMEM_SHARED((_CSZ,), jnp.float32),
    ],
)(_sc_counts_body)


def _gat_layer(h_in, C, mask, W, a_s_row, a_d_col, b_row):
    h = jnp.dot(h_in, W, preferred_element_type=jnp.float32)
    al_s = jax.lax.dot_general(
        a_s_row, h, (((1,), (1,)), ((), ())),
        preferred_element_type=jnp.float32)
    al_d = jnp.dot(h, a_d_col, preferred_element_type=jnp.float32)
    e = al_d + al_s
    e = jnp.where(e >= 0, e, 0.2 * e)
    em = jnp.where(mask, e, -1e30)
    m = jnp.max(em, axis=1, keepdims=True)
    p = jnp.exp(em - m) * C
    denom = jnp.sum(p, axis=1, keepdims=True)
    P = p / (denom + 1e-16)
    return jnp.dot(P, h, preferred_element_type=jnp.float32) + b_row


def _gat3_kernel(c_ref, x_ref,
                 w1_ref, as1_ref, ad1_ref, b1_ref,
                 w2_ref, as2_ref, ad2_ref, b2_ref,
                 w3_ref, as3_ref, ad3_ref, b3_ref,
                 out_ref):
    rr = jax.lax.broadcasted_iota(jnp.int32, (N, N), 0)
    cc = jax.lax.broadcasted_iota(jnp.int32, (N, N), 1)
    C = c_ref[0] + c_ref[1] + (rr == cc).astype(jnp.float32)
    mask = C > 0
    h = x_ref[...]
    h = _gat_layer(h, C, mask, w1_ref[...], as1_ref[...], ad1_ref[...],
                   b1_ref[...])
    h = _gat_layer(h, C, mask, w2_ref[...], as2_ref[...], ad2_ref[...],
                   b2_ref[...])
    h = _gat_layer(h, C, mask, w3_ref[...], as3_ref[...], ad3_ref[...],
                   b3_ref[...])
    out_ref[...] = jnp.maximum(h[:, :F], 0.0)


def _run_gat3(counts2, xp, layer_params):
    flat = []
    for (Wp, a_s, a_d, b) in layer_params:
        flat += [Wp, a_s, a_d, b]
    return pl.pallas_call(
        _gat3_kernel,
        out_shape=jax.ShapeDtypeStruct((N, F), jnp.float32),
    )(counts2, xp, *flat)


_K1 = 3840
_N1 = 512
_NK1 = (N * F) // _K1
_NN1 = HIDDEN // _N1


def _mv1_kernel(y_ref, w_ref, b_ref, o_ref):
    k = pl.program_id(1)
    part = jnp.dot(y_ref[...], w_ref[...], preferred_element_type=jnp.float32)

    @pl.when(k == 0)
    def _():
        o_ref[...] = part

    @pl.when(k > 0)
    def _():
        o_ref[...] += part

    @pl.when(k == _NK1 - 1)
    def _():
        o_ref[...] = jnp.maximum(o_ref[...] + b_ref[...], 0.0)


def _run_mv1(y0, W4, b4):
    return pl.pallas_call(
        _mv1_kernel,
        grid=(_NN1, _NK1),
        in_specs=[
            pl.BlockSpec((1, _K1), lambda n, k: (0, k)),
            pl.BlockSpec((_K1, _N1), lambda n, k: (k, n)),
            pl.BlockSpec((1, _N1), lambda n, k: (0, n)),
        ],
        out_specs=pl.BlockSpec((1, _N1), lambda n, k: (0, n)),
        out_shape=jax.ShapeDtypeStruct((1, HIDDEN), jnp.float32),
    )(y0, W4, b4)


_N2 = 1920
_NN2 = OUT_DIM // _N2


def _mv2_kernel(y_ref, w_ref, b_ref, o_ref):
    o_ref[...] = (
        jnp.dot(y_ref[...], w_ref[...], preferred_element_type=jnp.float32)
        + b_ref[...])


def _run_mv2(y1, W5, b5):
    return pl.pallas_call(
        _mv2_kernel,
        grid=(_NN2,),
        in_specs=[
            pl.BlockSpec((1, HIDDEN), lambda j: (0, 0)),
            pl.BlockSpec((HIDDEN, _N2), lambda j: (0, j)),
            pl.BlockSpec((1, _N2), lambda j: (0, j)),
        ],
        out_specs=pl.BlockSpec((1, _N2), lambda j: (0, j)),
        out_shape=jax.ShapeDtypeStruct((1, OUT_DIM), jnp.float32),
    )(y1, W5, b5)


def _pad_w(W):
    return jnp.pad(W, ((0, 1), (0, 1)))


def kernel(x, edge_index, W1, a_src1, a_dst1, b1, W2, a_src2, a_dst2, b2,
           W3, a_src3, a_dst3, b3, W4, b4, W5, b5):
    ei = edge_index.astype(jnp.int32)
    counts2 = _sc_counts(ei[0], ei[1]).reshape(_NC, N, N)

    xp = jnp.pad(x, ((0, 0), (0, 1)))
    layer_params = []
    for (W, a_s, a_d, b) in ((W1, a_src1, a_dst1, b1),
                             (W2, a_src2, a_dst2, b2),
                             (W3, a_src3, a_dst3, b3)):
        layer_params.append((
            _pad_w(W),
            jnp.pad(a_s, (0, 1)).reshape(1, N),
            jnp.pad(a_d, (0, 1)).reshape(N, 1),
            jnp.pad(b, (0, 1)).reshape(1, N),
        ))

    g3r = _run_gat3(counts2, xp, layer_params)
    y0 = g3r.reshape(1, N * F)
    y1 = _run_mv1(y0, W4, b4.reshape(1, HIDDEN))
    y2 = _run_mv2(y1, W5, b5.reshape(1, OUT_DIM))
    return y2.reshape(OUT_DIM)


# speedup vs baseline: 1.0199x; 1.0038x over previous
import functools

import jax
import jax.numpy as jnp
from jax import lax
from jax.experimental import pallas as pl
from jax.experimental.pallas import tpu as pltpu
from jax.experimental.pallas import tpu_sc as plsc

N = 256
F = 255
E = 65536
HIDDEN = 2048
OUT_DIM = 32640


_NC = 2
_NS = 16
_NW = _NC * _NS
_EPW = E // _NW
_CSZ = N * N
_ZSL = _CSZ // _NS
_NROW = _EPW // 128


def _sc_counts_body(src_hbm, dst_hbm, out_hbm,
                    src_v, dst_v, idx_v, val_v, zer_v, c_sh):
    cid = lax.axis_index("c")
    sid = lax.axis_index("s")
    wid = sid * _NC + cid
    base = wid * _EPW
    pltpu.sync_copy(src_hbm.at[pl.ds(base, _EPW)], src_v)
    pltpu.sync_copy(dst_hbm.at[pl.ds(base, _EPW)], dst_v)

    zeros = jnp.zeros((16,), jnp.float32)
    ones = jnp.ones((16,), jnp.float32)
    for j in range(_ZSL // 16):
        zer_v[pl.ds(j * 16, 16)] = zeros
    for k in range(8):
        val_v[pl.ds(k * 16, 16)] = ones
    for j in range(_NROW):
        for k in range(8):
            s = pl.ds((j * 8 + k) * 16, 16)
            idx_v[j, pl.ds(k * 16, 16)] = dst_v[s] * 256 + src_v[s]

    pltpu.sync_copy(zer_v, c_sh.at[pl.ds(sid * _ZSL, _ZSL)])
    plsc.subcore_barrier()
    for j in range(_NROW):
        pltpu.sync_copy(val_v, c_sh.at[idx_v.at[j]], add=True)
    plsc.subcore_barrier()
    pltpu.sync_copy(c_sh.at[pl.ds(sid * _ZSL, _ZSL)],
                    out_hbm.at[cid, pl.ds(sid * _ZSL, _ZSL)])


_sc_counts = functools.partial(
    pl.kernel,
    out_type=jax.ShapeDtypeStruct((_NC, _CSZ), jnp.float32),
    mesh=plsc.VectorSubcoreMesh(core_axis_name="c", subcore_axis_name="s"),
    scratch_types=[
        pltpu.VMEM((_EPW,), jnp.int32),
        pltpu.VMEM((_EPW,), jnp.int32),
        pltpu.VMEM((_NROW, 128), jnp.int32),
        pltpu.VMEM((128,), jnp.float32),
        pltpu.VMEM((_ZSL,), jnp.float32),
        pltpu.VMEM_SHARED((_CSZ,), jnp.float32),
    ],
)(_sc_counts_body)


def _gat_layer(h_in, C, mask, W, a_s_row, a_d_col, b_row):
    h = jnp.dot(h_in, W, preferred_element_type=jnp.float32)
    al_s = jax.lax.dot_general(
        a_s_row, h, (((1,), (1,)), ((), ())),
        preferred_element_type=jnp.float32)
    al_d = jnp.dot(h, a_d_col, preferred_element_type=jnp.float32)
    e = al_d + al_s
    e = jnp.where(e >= 0, e, 0.2 * e)
    em = jnp.where(mask, e, -1e30)
    m = jnp.max(em, axis=1, keepdims=True)
    p = jnp.exp(em - m) * C
    denom = jnp.sum(p, axis=1, keepdims=True)
    P = p / (denom + 1e-16)
    return jnp.dot(P, h, preferred_element_type=jnp.float32) + b_row


def _gat3_kernel(c_ref, x_ref,
                 w1_ref, as1_ref, ad1_ref, b1_ref,
                 w2_ref, as2_ref, ad2_ref, b2_ref,
                 w3_ref, as3_ref, ad3_ref, b3_ref,
                 out_ref):
    rr = jax.lax.broadcasted_iota(jnp.int32, (N, N), 0)
    cc = jax.lax.broadcasted_iota(jnp.int32, (N, N), 1)
    C = c_ref[0] + c_ref[1] + (rr == cc).astype(jnp.float32)
    mask = C > 0
    h = x_ref[...]
    h = _gat_layer(h, C, mask, w1_ref[...], as1_ref[...], ad1_ref[...],
                   b1_ref[...])
    h = _gat_layer(h, C, mask, w2_ref[...], as2_ref[...], ad2_ref[...],
                   b2_ref[...])
    h = _gat_layer(h, C, mask, w3_ref[...], as3_ref[...], ad3_ref[...],
                   b3_ref[...])
    out_ref[...] = jnp.maximum(h[:, :F], 0.0)


def _run_gat3(counts2, xp, layer_params):
    flat = []
    for (Wp, a_s, a_d, b) in layer_params:
        flat += [Wp, a_s, a_d, b]
    return pl.pallas_call(
        _gat3_kernel,
        out_shape=jax.ShapeDtypeStruct((N, F), jnp.float32),
    )(counts2, xp, *flat)


_K1 = 1920
_NK1 = (N * F) // _K1


def _mv1_kernel(y_ref, w_ref, b_ref, o_ref):
    k = pl.program_id(0)
    part = jnp.dot(y_ref[...], w_ref[...], preferred_element_type=jnp.float32)

    @pl.when(k == 0)
    def _():
        o_ref[...] = part

    @pl.when(k > 0)
    def _():
        o_ref[...] += part

    @pl.when(k == _NK1 - 1)
    def _():
        o_ref[...] = jnp.maximum(o_ref[...] + b_ref[...], 0.0)


def _run_mv1(y0, W4, b4):
    return pl.pallas_call(
        _mv1_kernel,
        grid=(_NK1,),
        in_specs=[
            pl.BlockSpec((1, _K1), lambda k: (0, k)),
            pl.BlockSpec((_K1, HIDDEN), lambda k: (k, 0)),
            pl.BlockSpec((1, HIDDEN), lambda k: (0, 0)),
        ],
        out_specs=pl.BlockSpec((1, HIDDEN), lambda k: (0, 0)),
        out_shape=jax.ShapeDtypeStruct((1, HIDDEN), jnp.float32),
    )(y0, W4, b4)


_K2 = 128
_NK2 = HIDDEN // _K2


def _mv2_kernel(y_ref, w_ref, b_ref, o_ref):
    k = pl.program_id(0)
    part = jnp.dot(y_ref[...], w_ref[...], preferred_element_type=jnp.float32)

    @pl.when(k == 0)
    def _():
        o_ref[...] = part + b_ref[...]

    @pl.when(k > 0)
    def _():
        o_ref[...] += part


def _run_mv2(y1, W5, b5):
    return pl.pallas_call(
        _mv2_kernel,
        grid=(_NK2,),
        in_specs=[
            pl.BlockSpec((1, _K2), lambda k: (0, k)),
            pl.BlockSpec((_K2, OUT_DIM), lambda k: (k, 0)),
            pl.BlockSpec((1, OUT_DIM), lambda k: (0, 0)),
        ],
        out_specs=pl.BlockSpec((1, OUT_DIM), lambda k: (0, 0)),
        out_shape=jax.ShapeDtypeStruct((1, OUT_DIM), jnp.float32),
    )(y1, W5, b5)


def _pad_w(W):
    return jnp.pad(W, ((0, 1), (0, 1)))


def kernel(x, edge_index, W1, a_src1, a_dst1, b1, W2, a_src2, a_dst2, b2,
           W3, a_src3, a_dst3, b3, W4, b4, W5, b5):
    ei = edge_index.astype(jnp.int32)
    counts2 = _sc_counts(ei[0], ei[1]).reshape(_NC, N, N)

    xp = jnp.pad(x, ((0, 0), (0, 1)))
    layer_params = []
    for (W, a_s, a_d, b) in ((W1, a_src1, a_dst1, b1),
                             (W2, a_src2, a_dst2, b2),
                             (W3, a_src3, a_dst3, b3)):
        layer_params.append((
            _pad_w(W),
            jnp.pad(a_s, (0, 1)).reshape(1, N),
            jnp.pad(a_d, (0, 1)).reshape(N, 1),
            jnp.pad(b, (0, 1)).reshape(1, N),
        ))

    g3r = _run_gat3(counts2, xp, layer_params)
    y0 = g3r.reshape(1, N * F)
    y1 = _run_mv1(y0, W4, b4.reshape(1, HIDDEN))
    y2 = _run_mv2(y1, W5, b5.reshape(1, OUT_DIM))
    return y2.reshape(OUT_DIM)
